# Initial kernel scaffold; baseline (speedup 1.0000x reference)
import jax
import jax.numpy as jnp
from jax.experimental import pallas as pl


def kernel(points, voxel_coords):
    raise NotImplementedError("write your pallas kernel here")


# speedup vs baseline: 25.6439x; 25.6439x over previous
import functools

import jax
import jax.numpy as jnp
from jax import lax
from jax.experimental import pallas as pl
from jax.experimental.pallas import tpu as pltpu
from jax.experimental.pallas import tpu_sc as plsc

NXY = 512
NPIX = NXY * NXY
N_POINTS = 500000

GROUPS = 8
QUARTS = 4
QSIZE = NPIX // QUARTS
BLK = 3136
BLKS = 20
CHUNK = BLK * BLKS
N_PAD = CHUNK * GROUPS
PAD = N_PAD - N_POINTS

_mesh = plsc.VectorSubcoreMesh(core_axis_name="c", subcore_axis_name="s")


@functools.partial(
    pl.kernel,
    out_type=jax.ShapeDtypeStruct((GROUPS, QUARTS, QSIZE), jnp.int32),
    mesh=_mesh,
    compiler_params=pltpu.CompilerParams(needs_layout_passes=False),
    scratch_types=[
        pltpu.VMEM((QSIZE,), jnp.int32),
        pltpu.VMEM((BLK,), jnp.int32),
        pltpu.VMEM((BLK,), jnp.int32),
        pltpu.VMEM((BLK,), jnp.int32),
        pltpu.VMEM((BLK,), jnp.float32),
        pltpu.VMEM((16,), jnp.int32),
        pltpu.VMEM((16,), jnp.int32),
    ],
)
def _scatter_max_kernel(c0_hbm, c1_hbm, c2_hbm, dy_hbm, part_hbm,
                        canvas, b0, b1, b2, bd, tmpk, tmpv):
    cid = lax.axis_index("c")
    sid = lax.axis_index("s")
    wid = cid * 16 + sid
    g = wid // QUARTS
    q = wid % QUARTS
    lo = q * QSIZE

    zeros16 = jnp.zeros((16,), jnp.int32)

    def _zero(i, carry):
        canvas[pl.ds(pl.multiple_of(i * 16, 16), 16)] = zeros16
        return carry

    lax.fori_loop(0, QSIZE // 16, _zero, 0)

    iota = lax.iota(jnp.int32, 16)

    def _vec(j, carry):
        o = pl.multiple_of(j * 16, 16)
        c0v = b0[pl.ds(o, 16)]
        c1v = b1[pl.ds(o, 16)]
        c2v = b2[pl.ds(o, 16)]
        dyv = bd[pl.ds(o, 16)]
        idx = c1v * NXY + c2v
        inr = (idx >= lo) & (idx < lo + QSIZE)
        li = jnp.where(inr, idx - lo, 0)
        bit = (jnp.abs(dyv) > 0.1).astype(jnp.int32)
        v = jnp.where(inr, c0v * 2 + 2 + bit, 0)
        sk, sv = plsc.sort_key_val(li, v)
        tmpk[...] = sk
        for d in (1, 2, 4, 8):
            tmpv[...] = sv
            perm = jnp.maximum(iota - d, 0)
            pk = plsc.load_gather(tmpk, [perm])
            pv = plsc.load_gather(tmpv, [perm])
            sv = jnp.where((pk == sk) & (iota >= d), jnp.maximum(sv, pv), sv)
        nxt = plsc.load_gather(tmpk, [jnp.minimum(iota + 1, 15)])
        is_last = (sk != nxt) | (iota == 15)
        old = plsc.load_gather(canvas, [sk])
        plsc.store_scatter(canvas, [sk], jnp.maximum(old, sv), mask=is_last)
        return carry

    def _block(blk, carry):
        base = pl.multiple_of(g * CHUNK + blk * BLK, 64)
        pltpu.sync_copy(c0_hbm.at[pl.ds(base, BLK)], b0)
        pltpu.sync_copy(c1_hbm.at[pl.ds(base, BLK)], b1)
        pltpu.sync_copy(c2_hbm.at[pl.ds(base, BLK)], b2)
        pltpu.sync_copy(dy_hbm.at[pl.ds(base, BLK)], bd)
        lax.fori_loop(0, BLK // 16, _vec, 0)
        return carry

    lax.fori_loop(0, BLKS, _block, 0)

    pltpu.sync_copy(canvas, part_hbm.at[g, q])


def _merge_body(p_ref, o_ref):
    m = jnp.max(p_ref[...], axis=0)
    o_ref[...] = (m & 1).astype(jnp.float32)


_merge = pl.pallas_call(
    _merge_body,
    grid=(64,),
    in_specs=[pl.BlockSpec((GROUPS, 8, NXY), lambda i: (0, i, 0))],
    out_specs=pl.BlockSpec((8, NXY), lambda i: (i, 0)),
    out_shape=jax.ShapeDtypeStruct((NXY, NXY), jnp.float32),
)


def kernel(points, voxel_coords):
    c = voxel_coords.astype(jnp.int32)
    c0 = jnp.pad(c[:, 0], (0, PAD))
    c1 = jnp.pad(c[:, 1], (0, PAD), constant_values=NXY)
    c2 = jnp.pad(c[:, 2], (0, PAD))
    dy = jnp.pad(points[:, 4], (0, PAD))
    parts = _scatter_max_kernel(c0, c1, c2, dy)
    img = _merge(parts.reshape(GROUPS, NXY, NXY))
    return img.reshape(1, 1, NXY, NXY)


# speedup vs baseline: 33.6784x; 1.3133x over previous
import functools

import jax
import jax.numpy as jnp
from jax import lax
from jax.experimental import pallas as pl
from jax.experimental.pallas import tpu as pltpu
from jax.experimental.pallas import tpu_sc as plsc

NXY = 512
NPIX = NXY * NXY
N_POINTS = 500000

GROUPS = 8
QUARTS = 4
QSIZE = NPIX // QUARTS
BLK = 3136
BLKS = 20
CHUNK = BLK * BLKS
N_PAD = CHUNK * GROUPS
PAD = N_PAD - N_POINTS

_mesh = plsc.VectorSubcoreMesh(core_axis_name="c", subcore_axis_name="s")


@functools.partial(
    pl.kernel,
    out_type=jax.ShapeDtypeStruct((GROUPS, QUARTS, QSIZE), jnp.int32),
    mesh=_mesh,
    compiler_params=pltpu.CompilerParams(needs_layout_passes=False),
    scratch_types=[
        pltpu.VMEM((QSIZE,), jnp.int32),
        pltpu.VMEM((BLK,), jnp.int32),
        pltpu.VMEM((BLK,), jnp.int32),
        pltpu.VMEM((BLK,), jnp.int32),
        pltpu.VMEM((BLK,), jnp.float32),
        pltpu.VMEM((BLK,), jnp.int32),
        pltpu.VMEM((BLK,), jnp.int32),
        pltpu.VMEM((BLK,), jnp.int32),
        pltpu.VMEM((BLK,), jnp.float32),
        pltpu.SemaphoreType.DMA,
        pltpu.SemaphoreType.DMA,
    ],
)
def _scatter_max_kernel(c0_hbm, c1_hbm, c2_hbm, dy_hbm, part_hbm,
                        canvas, b0a, b1a, b2a, bda, b0b, b1b, b2b, bdb,
                        sem0, sem1):
    cid = lax.axis_index("c")
    sid = lax.axis_index("s")
    wid = cid * 16 + sid
    g = wid // QUARTS
    q = wid % QUARTS
    lo = q * QSIZE

    bufs = ((b0a, b1a, b2a, bda), (b0b, b1b, b2b, bdb))
    sems = (sem0, sem1)

    def _start(blk, buf):
        base = pl.multiple_of(g * CHUNK + blk * BLK, 16)
        s = sems[buf]
        t0, t1, t2, td = bufs[buf]
        return (
            pltpu.async_copy(c0_hbm.at[pl.ds(base, BLK)], t0, s),
            pltpu.async_copy(c1_hbm.at[pl.ds(base, BLK)], t1, s),
            pltpu.async_copy(c2_hbm.at[pl.ds(base, BLK)], t2, s),
            pltpu.async_copy(dy_hbm.at[pl.ds(base, BLK)], td, s),
        )

    pend = _start(0, 0)

    zeros16 = jnp.zeros((16,), jnp.int32)

    def _zero(i, carry):
        o = pl.multiple_of(i * 128, 128)
        for u in range(8):
            canvas[pl.ds(o + u * 16, 16)] = zeros16
        return carry

    lax.fori_loop(0, QSIZE // 128, _zero, 0)

    for blk in range(BLKS):
        buf = blk & 1
        for h in pend:
            h.wait()
        if blk + 1 < BLKS:
            pend = _start(blk + 1, (blk + 1) & 1)

        t0, t1, t2, td = bufs[buf]

        def _vec(j, carry, t0=t0, t1=t1, t2=t2, td=td):
            o = pl.multiple_of(j * 16, 16)
            c0v = t0[pl.ds(o, 16)]
            c1v = t1[pl.ds(o, 16)]
            c2v = t2[pl.ds(o, 16)]
            dyv = td[pl.ds(o, 16)]
            idx = c1v * NXY + c2v
            inr = (idx >= lo) & (idx < lo + QSIZE)
            li = jnp.where(inr, idx - lo, 0)
            bit = (jnp.abs(dyv) > 0.1).astype(jnp.int32)
            v = jnp.where(inr, c0v * 2 + 2 + bit, 0)
            old = plsc.load_gather(canvas, [li])
            plsc.store_scatter(canvas, [li], jnp.maximum(old, v))

            def _cond(m):
                return jnp.any(m)

            def _fix(m):
                cur = plsc.load_gather(canvas, [li])
                need = m & (cur < v)
                plsc.store_scatter(canvas, [li], jnp.maximum(cur, v), mask=need)
                return plsc.load_gather(canvas, [li]) < v

            m0 = plsc.load_gather(canvas, [li]) < v
            lax.while_loop(_cond, _fix, m0)
            return carry

        lax.fori_loop(0, BLK // 16, _vec, 0)

    pltpu.sync_copy(canvas, part_hbm.at[g, q])


def _merge_body(p_ref, o_ref):
    m = jnp.max(p_ref[...], axis=0)
    o_ref[...] = (m & 1).astype(jnp.float32)


_merge = pl.pallas_call(
    _merge_body,
    grid=(64,),
    in_specs=[pl.BlockSpec((GROUPS, 8, NXY), lambda i: (0, i, 0))],
    out_specs=pl.BlockSpec((8, NXY), lambda i: (i, 0)),
    out_shape=jax.ShapeDtypeStruct((NXY, NXY), jnp.float32),
)


def kernel(points, voxel_coords):
    c = voxel_coords.astype(jnp.int32)
    c0 = jnp.pad(c[:, 0], (0, PAD))
    c1 = jnp.pad(c[:, 1], (0, PAD), constant_values=NXY)
    c2 = jnp.pad(c[:, 2], (0, PAD))
    dy = jnp.pad(points[:, 4], (0, PAD))
    parts = _scatter_max_kernel(c0, c1, c2, dy)
    img = _merge(parts.reshape(GROUPS, NXY, NXY))
    return img.reshape(1, 1, NXY, NXY)


# speedup vs baseline: 34.6180x; 1.0279x over previous
import functools

import jax
import jax.numpy as jnp
from jax import lax
from jax.experimental import pallas as pl
from jax.experimental.pallas import tpu as pltpu
from jax.experimental.pallas import tpu_sc as plsc

NXY = 512
NPIX = NXY * NXY
N_POINTS = 500000

GROUPS = 8
QUARTS = 4
QSIZE = NPIX // QUARTS
BLK = 3136
BLKS = 20
CHUNK = BLK * BLKS
N_PAD = CHUNK * GROUPS
PAD = N_PAD - N_POINTS

_mesh = plsc.VectorSubcoreMesh(core_axis_name="c", subcore_axis_name="s")


@functools.partial(
    pl.kernel,
    out_type=jax.ShapeDtypeStruct((GROUPS, QUARTS, QSIZE), jnp.int32),
    mesh=_mesh,
    compiler_params=pltpu.CompilerParams(needs_layout_passes=False),
    scratch_types=[
        pltpu.VMEM((QSIZE,), jnp.int32),
        pltpu.VMEM((BLK,), jnp.int32),
        pltpu.VMEM((BLK,), jnp.int32),
        pltpu.VMEM((BLK,), jnp.int32),
        pltpu.VMEM((BLK,), jnp.int32),
        pltpu.VMEM((BLK,), jnp.int32),
        pltpu.VMEM((BLK,), jnp.int32),
        pltpu.VMEM((BLK,), jnp.int32),
        pltpu.VMEM((BLK,), jnp.int32),
        pltpu.SemaphoreType.DMA,
        pltpu.SemaphoreType.DMA,
    ],
)
def _scatter_max_kernel(cols_hbm, part_hbm,
                        canvas, b0a, b1a, b2a, bda, b0b, b1b, b2b, bdb,
                        sem0, sem1):
    cid = lax.axis_index("c")
    sid = lax.axis_index("s")
    wid = cid * 16 + sid
    g = wid // QUARTS
    q = wid % QUARTS
    lo = q * QSIZE

    bufs = ((b0a, b1a, b2a, bda), (b0b, b1b, b2b, bdb))
    sems = (sem0, sem1)

    def _start(blk, buf):
        base = pl.multiple_of(g * CHUNK + blk * BLK, 16)
        s = sems[buf]
        t0, t1, t2, td = bufs[buf]
        return (
            pltpu.async_copy(cols_hbm.at[pl.ds(base, BLK)], t0, s),
            pltpu.async_copy(cols_hbm.at[pl.ds(base + N_PAD, BLK)], t1, s),
            pltpu.async_copy(cols_hbm.at[pl.ds(base + 2 * N_PAD, BLK)], t2, s),
            pltpu.async_copy(cols_hbm.at[pl.ds(base + 3 * N_PAD, BLK)], td, s),
        )

    pend = _start(0, 0)

    zeros16 = jnp.zeros((16,), jnp.int32)

    def _zero(i, carry):
        o = pl.multiple_of(i * 128, 128)
        for u in range(8):
            canvas[pl.ds(o + u * 16, 16)] = zeros16
        return carry

    lax.fori_loop(0, QSIZE // 128, _zero, 0)

    for blk in range(BLKS):
        buf = blk & 1
        for h in pend:
            h.wait()
        if blk + 1 < BLKS:
            pend = _start(blk + 1, (blk + 1) & 1)

        t0, t1, t2, td = bufs[buf]

        def _vec(j, carry, t0=t0, t1=t1, t2=t2, td=td):
            o = pl.multiple_of(j * 16, 16)
            c0v = t0[pl.ds(o, 16)]
            c1v = t1[pl.ds(o, 16)]
            c2v = t2[pl.ds(o, 16)]
            dyv = plsc.bitcast(td[pl.ds(o, 16)], jnp.float32)
            idx = c1v * NXY + c2v
            inr = (idx >= lo) & (idx < lo + QSIZE)
            li = jnp.where(inr, idx - lo, 0)
            bit = (jnp.abs(dyv) > 0.1).astype(jnp.int32)
            v = jnp.where(inr, c0v * 2 + 2 + bit, 0)
            old = plsc.load_gather(canvas, [li])
            plsc.store_scatter(canvas, [li], jnp.maximum(old, v))

            def _cond(m):
                return jnp.any(m)

            def _fix(m):
                cur = plsc.load_gather(canvas, [li])
                need = m & (cur < v)
                plsc.store_scatter(canvas, [li], jnp.maximum(cur, v), mask=need)
                return plsc.load_gather(canvas, [li]) < v

            m0 = plsc.load_gather(canvas, [li]) < v
            lax.while_loop(_cond, _fix, m0)
            return carry

        lax.fori_loop(0, BLK // 16, _vec, 0)

    pltpu.sync_copy(canvas, part_hbm.at[g, q])


def _merge_body(p_ref, o_ref):
    m = jnp.max(p_ref[...], axis=0)
    o_ref[...] = (m & 1).astype(jnp.float32)


_merge = pl.pallas_call(
    _merge_body,
    grid=(64,),
    in_specs=[pl.BlockSpec((GROUPS, 8, NXY), lambda i: (0, i, 0))],
    out_specs=pl.BlockSpec((8, NXY), lambda i: (i, 0)),
    out_shape=jax.ShapeDtypeStruct((NXY, NXY), jnp.float32),
)


def kernel(points, voxel_coords):
    c = voxel_coords.astype(jnp.int32)
    c0 = jnp.pad(c[:, 0], (0, PAD))
    c1 = jnp.pad(c[:, 1], (0, PAD), constant_values=NXY)
    c2 = jnp.pad(c[:, 2], (0, PAD))
    dy = jnp.pad(points[:, 4], (0, PAD)).view(jnp.int32)
    cols = jnp.concatenate([c0, c1, c2, dy])
    parts = _scatter_max_kernel(cols)
    img = _merge(parts.reshape(GROUPS, NXY, NXY))
    return img.reshape(1, 1, NXY, NXY)


# speedup vs baseline: 56.4305x; 1.6301x over previous
import functools

import jax
import jax.numpy as jnp
from jax import lax
from jax.experimental import pallas as pl
from jax.experimental.pallas import tpu as pltpu
from jax.experimental.pallas import tpu_sc as plsc

NXY = 512
NPIX = NXY * NXY
N_POINTS = 500000

GROUPS = 8
QUARTS = 4
QSIZE = NPIX // QUARTS
VBITS = 11
BLK = 3968
BLKS = 16
CHUNK = BLK * BLKS
N_PAD = CHUNK * GROUPS
PAD = N_PAD - N_POINTS
UNROLL = 8

_mesh = plsc.VectorSubcoreMesh(core_axis_name="c", subcore_axis_name="s")


@functools.partial(
    pl.kernel,
    out_type=jax.ShapeDtypeStruct((GROUPS, QUARTS, QSIZE), jnp.int32),
    mesh=_mesh,
    compiler_params=pltpu.CompilerParams(needs_layout_passes=False),
    scratch_types=[
        pltpu.VMEM((QSIZE,), jnp.int32),
        pltpu.VMEM((BLK,), jnp.int32),
        pltpu.VMEM((BLK,), jnp.int32),
        pltpu.SemaphoreType.DMA,
        pltpu.SemaphoreType.DMA,
    ],
)
def _scatter_max_kernel(w_hbm, part_hbm, canvas, bufa, bufb, sem0, sem1):
    cid = lax.axis_index("c")
    sid = lax.axis_index("s")
    wid = cid * 16 + sid
    g = wid // QUARTS
    q = wid % QUARTS
    lo2 = q * (QSIZE << VBITS)
    hi2 = lo2 + (QSIZE << VBITS)

    bufs = (bufa, bufb)
    sems = (sem0, sem1)

    def _start(blk, buf):
        base = pl.multiple_of(g * CHUNK + blk * BLK, 16)
        return (pltpu.async_copy(w_hbm.at[pl.ds(base, BLK)], bufs[buf],
                                 sems[buf]),)

    pend = _start(0, 0)

    zeros16 = jnp.zeros((16,), jnp.int32)

    def _zero(i, carry):
        o = pl.multiple_of(i * 128, 128)
        for u in range(8):
            canvas[pl.ds(o + u * 16, 16)] = zeros16
        return carry

    lax.fori_loop(0, QSIZE // 128, _zero, 0)

    def _decode(tb, o):
        w = tb[pl.ds(o, 16)]
        msk = (w >= lo2) & (w < hi2)
        li = jnp.where(msk, (w - lo2) >> VBITS, 0)
        v = jnp.where(msk, w & ((1 << VBITS) - 1), 0)
        return li, v

    for blk in range(BLKS):
        buf = blk & 1
        for h in pend:
            h.wait()
        if blk + 1 < BLKS:
            pend = _start(blk + 1, (blk + 1) & 1)

        tb = bufs[buf]

        def _grp(t, carry, tb=tb):
            o0 = pl.multiple_of(t * (16 * UNROLL), 16 * UNROLL)
            m = None
            for u in range(UNROLL):
                li, v = _decode(tb, pl.multiple_of(o0 + u * 16, 16))
                old = plsc.load_gather(canvas, [li])
                plsc.store_scatter(canvas, [li], jnp.maximum(old, v))
                cur = plsc.load_gather(canvas, [li])
                need = cur < v
                m = need if m is None else m | need

            def _fc(mm):
                return jnp.any(mm)

            def _fb(mm):
                nm = None
                for u in range(UNROLL):
                    li, v = _decode(tb, pl.multiple_of(o0 + u * 16, 16))
                    cur = plsc.load_gather(canvas, [li])
                    plsc.store_scatter(canvas, [li], jnp.maximum(cur, v),
                                       mask=cur < v)
                    cur2 = plsc.load_gather(canvas, [li])
                    bad = cur2 < v
                    nm = bad if nm is None else nm | bad
                return nm

            lax.while_loop(_fc, _fb, m)
            return carry

        lax.fori_loop(0, BLK // (16 * UNROLL), _grp, 0)

    pltpu.sync_copy(canvas, part_hbm.at[g, q])


def _merge_body(p_ref, o_ref):
    m = jnp.max(p_ref[...], axis=0)
    o_ref[...] = (m & 1).astype(jnp.float32)


_merge = pl.pallas_call(
    _merge_body,
    grid=(64,),
    in_specs=[pl.BlockSpec((GROUPS, 8, NXY), lambda i: (0, i, 0))],
    out_specs=pl.BlockSpec((8, NXY), lambda i: (i, 0)),
    out_shape=jax.ShapeDtypeStruct((NXY, NXY), jnp.float32),
)


def kernel(points, voxel_coords):
    c = voxel_coords.astype(jnp.int32)
    bit = (jnp.abs(points[:, 4]) > 0.1).astype(jnp.int32)
    w = ((c[:, 1] * NXY + c[:, 2]) << VBITS) | (2 * c[:, 0] + 2 + bit)
    w = jnp.pad(w, (0, PAD), constant_values=NPIX << VBITS)
    parts = _scatter_max_kernel(w)
    img = _merge(parts.reshape(GROUPS, NXY, NXY))
    return img.reshape(1, 1, NXY, NXY)


# speedup vs baseline: 114.4995x; 2.0290x over previous
import functools

import jax
import jax.numpy as jnp
from jax import lax
from jax.experimental import pallas as pl
from jax.experimental.pallas import tpu as pltpu
from jax.experimental.pallas import tpu_sc as plsc

NXY = 512
NPIX = NXY * NXY
N_POINTS = 500000

NTILES = 32
RSIZE = NPIX // NTILES
VBITS = 11
BLK = 3968
BLKS = 8
CHUNK = BLK * BLKS
N_PAD = CHUNK * 16
PAD = N_PAD - N_POINTS
UNROLL = 8

_mesh = plsc.VectorSubcoreMesh(core_axis_name="c", subcore_axis_name="s")


@functools.partial(
    pl.kernel,
    out_type=jax.ShapeDtypeStruct((NPIX,), jnp.float32),
    mesh=_mesh,
    compiler_params=pltpu.CompilerParams(needs_layout_passes=False),
    scratch_types=[
        pltpu.VMEM((RSIZE,), jnp.int32),
        pltpu.VMEM((RSIZE,), jnp.float32),
        pltpu.VMEM((BLK,), jnp.int32),
        pltpu.VMEM((BLK,), jnp.int32),
        pltpu.SemaphoreType.DMA,
        pltpu.SemaphoreType.DMA,
    ],
)
def _scatter_max_kernel(w_hbm, out_hbm, canvas, outb, bufa, bufb, sem0, sem1):
    cid = lax.axis_index("c")
    sid = lax.axis_index("s")
    wid = cid * 16 + sid
    lo2 = wid * (RSIZE << VBITS)
    hi2 = lo2 + (RSIZE << VBITS)

    bufs = (bufa, bufb)
    sems = (sem0, sem1)

    def _start(blk, buf):
        base = pl.multiple_of(sid * CHUNK + blk * BLK, 16)
        return (pltpu.async_copy(w_hbm.at[pl.ds(base, BLK)], bufs[buf],
                                 sems[buf]),)

    pend = _start(0, 0)

    zeros16 = jnp.zeros((16,), jnp.int32)

    def _zero(i, carry):
        o = pl.multiple_of(i * 128, 128)
        for u in range(8):
            canvas[pl.ds(o + u * 16, 16)] = zeros16
        return carry

    lax.fori_loop(0, RSIZE // 128, _zero, 0)

    def _decode(tb, o):
        w = tb[pl.ds(o, 16)]
        msk = (w >= lo2) & (w < hi2)
        li = jnp.where(msk, (w - lo2) >> VBITS, 0)
        v = jnp.where(msk, w & ((1 << VBITS) - 1), 0)
        return li, v

    for blk in range(BLKS):
        buf = blk & 1
        for h in pend:
            h.wait()
        if blk + 1 < BLKS:
            pend = _start(blk + 1, (blk + 1) & 1)

        tb = bufs[buf]

        def _grp(t, carry, tb=tb):
            o0 = pl.multiple_of(t * (16 * UNROLL), 16 * UNROLL)
            m = None
            for u in range(UNROLL):
                li, v = _decode(tb, pl.multiple_of(o0 + u * 16, 16))
                old = plsc.load_gather(canvas, [li])
                plsc.store_scatter(canvas, [li], jnp.maximum(old, v))
                cur = plsc.load_gather(canvas, [li])
                need = cur < v
                m = need if m is None else m | need

            def _fc(mm):
                return jnp.any(mm)

            def _fb(mm):
                nm = None
                for u in range(UNROLL):
                    li, v = _decode(tb, pl.multiple_of(o0 + u * 16, 16))
                    cur = plsc.load_gather(canvas, [li])
                    plsc.store_scatter(canvas, [li], jnp.maximum(cur, v),
                                       mask=cur < v)
                    cur2 = plsc.load_gather(canvas, [li])
                    bad = cur2 < v
                    nm = bad if nm is None else nm | bad
                return nm

            lax.while_loop(_fc, _fb, m)
            return carry

        lax.fori_loop(0, BLK // (16 * UNROLL), _grp, 0)

    def _conv(i, carry):
        o = pl.multiple_of(i * 16, 16)
        outb[pl.ds(o, 16)] = (canvas[pl.ds(o, 16)] & 1).astype(jnp.float32)
        return carry

    lax.fori_loop(0, RSIZE // 16, _conv, 0)
    pltpu.sync_copy(outb, out_hbm.at[pl.ds(wid * RSIZE, RSIZE)])


def kernel(points, voxel_coords):
    c = voxel_coords.astype(jnp.int32)
    bit = (jnp.abs(points[:, 4]) > 0.1).astype(jnp.int32)
    w = ((c[:, 1] * NXY + c[:, 2]) << VBITS) | (2 * c[:, 0] + 2 + bit)
    w = jnp.pad(w, (0, PAD), constant_values=NPIX << VBITS)
    img = _scatter_max_kernel(w)
    return img.reshape(1, 1, NXY, NXY)
